# baseline (speedup 1.0000x reference)
import functools

import jax
import jax.numpy as jnp
from jax import lax
from jax.experimental import pallas as pl
from jax.experimental.pallas import tpu as pltpu
from jax.experimental.pallas import tpu_sc as plsc

N = 10000
E = 320000
D = 128
NC, NS = 2, 16
NW = NC * NS
NPAD = 10016
CH = 128
NCH = 80
EPT_PAD = NCH * CH
E_PAD = EPT_PAD * NW
NROWS_T = N // NS
NZROWS_T = NPAD // NS
DEPTH = 2
CPT = (124, 36)
CBASE = (0, NS * CPT[0])
CPT_MAX = max(CPT)

_sc_params = pltpu.CompilerParams(use_tc_tiling_on_sc=False,
                                  needs_layout_passes=False)


@functools.lru_cache(maxsize=None)
def _sc_kernels():
    mesh = plsc.VectorSubcoreMesh(core_axis_name="c", subcore_axis_name="s",
                                  num_cores=NC, num_subcores=NS)
    degrees = pl.kernel(
        _degrees_body,
        out_type=(
            jax.ShapeDtypeStruct((NW, NPAD), jnp.float32),
            jax.ShapeDtypeStruct((NW, NPAD), jnp.float32),
        ),
        mesh=mesh,
        compiler_params=_sc_params,
        scratch_types=[
            pltpu.VMEM((EPT_PAD,), jnp.int32),
            pltpu.VMEM((EPT_PAD,), jnp.int32),
            pltpu.VMEM((NPAD,), jnp.float32),
            pltpu.VMEM((NPAD,), jnp.float32),
        ],
    )
    aggregate = pl.kernel(
        _aggregate_body,
        out_type=jax.ShapeDtypeStruct((NC, N, D), jnp.float32),
        mesh=mesh,
        compiler_params=_sc_params,
        scratch_types=[
            pltpu.VMEM((CH,), jnp.int32),
            pltpu.VMEM((CH,), jnp.int32),
            pltpu.VMEM((CPT_MAX, CH), jnp.int32),
            pltpu.VMEM((CH, D), jnp.float32),
            pltpu.VMEM((CH, D), jnp.float32),
            pltpu.SemaphoreType.DMA,
            pltpu.SemaphoreType.DMA,
            pltpu.SemaphoreType.DMA,
            pltpu.SemaphoreType.DMA,
            pltpu.VMEM_SHARED((NPAD, D), jnp.float32),
        ],
    )
    return degrees, aggregate


def _degrees_body(src_hbm, dst_hbm, odeg_hbm, ideg_hbm, src_v, dst_v, oh_v, ih_v):
    c = lax.axis_index("c")
    s = lax.axis_index("s")
    t = s * NC + c
    zeros = jnp.zeros((16,), jnp.float32)

    def zbody(j, carry):
        oh_v[pl.ds(j * 16, 16)] = zeros
        ih_v[pl.ds(j * 16, 16)] = zeros
        return carry

    lax.fori_loop(0, NPAD // 16, zbody, 0)

    base = t * EPT_PAD
    pltpu.sync_copy(src_hbm.at[pl.ds(base, EPT_PAD)], src_v)
    pltpu.sync_copy(dst_hbm.at[pl.ds(base, EPT_PAD)], dst_v)

    ones = jnp.ones((16,), jnp.float32)

    def body(i, carry):
        si = src_v[pl.ds(i * 16, 16)]
        plsc.addupdate_scatter(oh_v, [si], ones)
        di = dst_v[pl.ds(i * 16, 16)]
        plsc.addupdate_scatter(ih_v, [di], ones)
        return carry

    lax.fori_loop(0, EPT_PAD // 16, body, 0)

    pltpu.sync_copy(oh_v, odeg_hbm.at[t])
    pltpu.sync_copy(ih_v, ideg_hbm.at[t])


def _aggregate_body(feat_hbm, src_hbm, dst_hbm, zero_hbm, out_hbm,
                    si0, si1, didx, r0, r1, gs0, gs1, is0, is1, acc_sh):
    c = lax.axis_index("c")
    s = lax.axis_index("s")
    t = s * NC + c
    rows = (r0, r1)
    gsems = (gs0, gs1)
    sidx = (si0, si1)
    isems = (is0, is1)

    with jax.named_scope("acc_zero"):
        pltpu.sync_copy(zero_hbm.at[pl.ds(s * NZROWS_T, NZROWS_T)],
                        acc_sh.at[pl.ds(s * NZROWS_T, NZROWS_T)])
        plsc.subcore_barrier()

    def run_pipe(base, n):
        pltpu.sync_copy(dst_hbm.at[pl.ds(base, n)], didx.at[pl.ds(0, n)])
        for j in range(DEPTH):
            pltpu.sync_copy(src_hbm.at[base + j], sidx[j])
            pltpu.async_copy(feat_hbm.at[sidx[j]], rows[j], gsems[j])

        def step(gg, j, refill):
            pltpu.make_async_copy(feat_hbm.at[sidx[j]], rows[j],
                                  gsems[j]).wait()
            if refill:
                pltpu.async_copy(src_hbm.at[base + gg + DEPTH], sidx[j],
                                 isems[j])
            pltpu.sync_copy(rows[j], acc_sh.at[didx.at[gg]], add=True)
            if refill:
                pltpu.make_async_copy(src_hbm.at[0], sidx[j],
                                      isems[j]).wait()
                pltpu.async_copy(feat_hbm.at[sidx[j]], rows[j], gsems[j])

        def body(q, carry):
            g = q * DEPTH
            for j in range(DEPTH):
                step(g + j, j, True)
            return carry

        lax.fori_loop(0, n // DEPTH - 1, body, 0)
        for j in range(DEPTH):
            step(n - DEPTH + j, j, False)

    with jax.named_scope("pipe"):
        @pl.when(c == 0)
        def _():
            run_pipe(CBASE[0] + s * CPT[0], CPT[0])

        @pl.when(c == 1)
        def _():
            run_pipe(CBASE[1] + s * CPT[1], CPT[1])

        plsc.subcore_barrier()

    with jax.named_scope("acc_out"):
        pltpu.sync_copy(acc_sh.at[pl.ds(s * NROWS_T, NROWS_T)],
                        out_hbm.at[c, pl.ds(s * NROWS_T, NROWS_T)])


def _prep_body(feat_ref, od_ref, wmu_ref, wrho_ref, weps_ref,
               bmu_ref, brho_ref, beps_ref, fs_ref, w_ref, b_ref):
    od = jnp.sum(od_ref[...], axis=0)
    od = jnp.maximum(od[:N], 1.0)
    fs_ref[pl.ds(0, N), :] = feat_ref[...] * lax.rsqrt(od)[:, None]
    fs_ref[pl.ds(N, NPAD - N), :] = jnp.zeros((NPAD - N, D), jnp.float32)
    w_ref[...] = wmu_ref[...] + weps_ref[...] * jnp.log1p(jnp.exp(wrho_ref[...]))
    b_ref[...] = bmu_ref[...] + beps_ref[...] * jnp.log1p(jnp.exp(brho_ref[...]))


_prep = pl.pallas_call(
    _prep_body,
    out_shape=(
        jax.ShapeDtypeStruct((NPAD, D), jnp.float32),
        jax.ShapeDtypeStruct((D, D), jnp.float32),
        jax.ShapeDtypeStruct((1, D), jnp.float32),
    ),
)


def _finish_body(p_ref, id_ref, w_ref, b_ref, out_ref):
    agg = p_ref[0] + p_ref[1]
    idg = jnp.maximum(jnp.sum(id_ref[...], axis=0)[:N], 1.0)
    rst = agg * lax.rsqrt(idg)[:, None]
    out_ref[...] = (
        jnp.dot(rst, w_ref[...], preferred_element_type=jnp.float32)
        + b_ref[...]
    )


_finish = pl.pallas_call(
    _finish_body,
    out_shape=jax.ShapeDtypeStruct((N, D), jnp.float32),
)


def kernel(feat, edge_index, W_mu, W_rho, bias_mu, bias_rho, W_eps, bias_eps):
    ei = edge_index.astype(jnp.int32)
    pad = jnp.full((E_PAD - E,), N, jnp.int32)
    src_pad = jnp.concatenate([ei[0], pad])
    dst_pad = jnp.concatenate([ei[1], pad])

    degrees, aggregate = _sc_kernels()
    odeg_parts, ideg_parts = degrees(src_pad, dst_pad)
    feat_scaled, weight, bias = _prep(
        feat, odeg_parts, W_mu, W_rho, W_eps,
        bias_mu.reshape(1, D), bias_rho.reshape(1, D), bias_eps.reshape(1, D))
    zeros = jnp.zeros((NPAD, D), jnp.float32)
    partials = aggregate(feat_scaled,
                         src_pad.reshape(NW * NCH, CH),
                         dst_pad.reshape(NW * NCH, CH), zeros)
    return _finish(partials, ideg_parts, weight, bias)


# speedup vs baseline: 2.7344x; 2.7344x over previous
import functools

import jax
import jax.numpy as jnp
from jax import lax
from jax.experimental import pallas as pl
from jax.experimental.pallas import tpu as pltpu
from jax.experimental.pallas import tpu_sc as plsc

N = 10000
E = 320000
D = 128
NC, NS = 2, 16
NW = NC * NS
NPAD = 10016
CH = 128
NCH = 80
EPT_PAD = NCH * CH
E_PAD = EPT_PAD * NW
NROWS_T = N // NS
NZROWS_T = NPAD // NS
DEPTH = 2
CPT = (80, 80)
CBASE = (0, NS * CPT[0])
CPT_MAX = max(CPT)

_sc_params = pltpu.CompilerParams(use_tc_tiling_on_sc=False,
                                  needs_layout_passes=False)


@functools.lru_cache(maxsize=None)
def _sc_kernels():
    mesh = plsc.VectorSubcoreMesh(core_axis_name="c", subcore_axis_name="s",
                                  num_cores=NC, num_subcores=NS)
    degrees = pl.kernel(
        _degrees_body,
        out_type=(
            jax.ShapeDtypeStruct((NW, NPAD), jnp.float32),
            jax.ShapeDtypeStruct((NW, NPAD), jnp.float32),
        ),
        mesh=mesh,
        compiler_params=_sc_params,
        scratch_types=[
            pltpu.VMEM((EPT_PAD,), jnp.int32),
            pltpu.VMEM((EPT_PAD,), jnp.int32),
            pltpu.VMEM((NPAD,), jnp.float32),
            pltpu.VMEM((NPAD,), jnp.float32),
        ],
    )
    aggregate = pl.kernel(
        _aggregate_body,
        out_type=jax.ShapeDtypeStruct((NC, N, D), jnp.float32),
        mesh=mesh,
        compiler_params=_sc_params,
        scratch_types=[
            pltpu.VMEM((CH,), jnp.int32),
            pltpu.VMEM((CH,), jnp.int32),
            pltpu.VMEM((CPT_MAX, CH), jnp.int32),
            pltpu.VMEM((CH, D), jnp.float32),
            pltpu.VMEM((CH, D), jnp.float32),
            pltpu.SemaphoreType.DMA,
            pltpu.SemaphoreType.DMA,
            pltpu.SemaphoreType.DMA,
            pltpu.SemaphoreType.DMA,
            pltpu.VMEM_SHARED((NPAD, D), jnp.float32),
        ],
    )
    return degrees, aggregate


def _degrees_body(src_hbm, dst_hbm, odeg_hbm, ideg_hbm, src_v, dst_v, oh_v, ih_v):
    c = lax.axis_index("c")
    s = lax.axis_index("s")
    t = s * NC + c
    zeros = jnp.zeros((16,), jnp.float32)

    def zbody(j, carry):
        oh_v[pl.ds(j * 16, 16)] = zeros
        ih_v[pl.ds(j * 16, 16)] = zeros
        return carry

    lax.fori_loop(0, NPAD // 16, zbody, 0)

    base = t * EPT_PAD
    pltpu.sync_copy(src_hbm.at[pl.ds(base, EPT_PAD)], src_v)
    pltpu.sync_copy(dst_hbm.at[pl.ds(base, EPT_PAD)], dst_v)

    ones = jnp.ones((16,), jnp.float32)

    def body(i, carry):
        si = src_v[pl.ds(i * 16, 16)]
        plsc.addupdate_scatter(oh_v, [si], ones)
        di = dst_v[pl.ds(i * 16, 16)]
        plsc.addupdate_scatter(ih_v, [di], ones)
        return carry

    lax.fori_loop(0, EPT_PAD // 16, body, 0)

    pltpu.sync_copy(oh_v, odeg_hbm.at[t])
    pltpu.sync_copy(ih_v, ideg_hbm.at[t])


def _aggregate_body(feat_hbm, src_hbm, dst_hbm, zero_hbm, out_hbm,
                    si0, si1, didx, r0, r1, gs0, gs1, is0, is1, acc_sh):
    c = lax.axis_index("c")
    s = lax.axis_index("s")
    t = s * NC + c
    rows = (r0, r1)
    gsems = (gs0, gs1)
    sidx = (si0, si1)
    isems = (is0, is1)

    with jax.named_scope("acc_zero"):
        pltpu.sync_copy(zero_hbm.at[pl.ds(s * NZROWS_T, NZROWS_T)],
                        acc_sh.at[pl.ds(s * NZROWS_T, NZROWS_T)])
        plsc.subcore_barrier()

    def run_pipe(base, n):
        pltpu.sync_copy(dst_hbm.at[pl.ds(base, n)], didx.at[pl.ds(0, n)])
        for j in range(DEPTH):
            pltpu.sync_copy(src_hbm.at[base + j], sidx[j])
            pltpu.async_copy(feat_hbm.at[sidx[j]], rows[j], gsems[j])

        def step(gg, j, refill):
            pltpu.make_async_copy(feat_hbm.at[sidx[j]], rows[j],
                                  gsems[j]).wait()
            if refill:
                pltpu.async_copy(src_hbm.at[base + gg + DEPTH], sidx[j],
                                 isems[j])
            pltpu.sync_copy(rows[j], acc_sh.at[didx.at[gg]], add=True)
            if refill:
                pltpu.make_async_copy(src_hbm.at[0], sidx[j],
                                      isems[j]).wait()
                pltpu.async_copy(feat_hbm.at[sidx[j]], rows[j], gsems[j])

        def body(q, carry):
            g = q * DEPTH
            for j in range(DEPTH):
                step(g + j, j, True)
            return carry

        lax.fori_loop(0, n // DEPTH - 1, body, 0)
        for j in range(DEPTH):
            step(n - DEPTH + j, j, False)

    with jax.named_scope("pipe"):
        @pl.when(c == 0)
        def _():
            run_pipe(CBASE[0] + s * CPT[0], CPT[0])

        @pl.when(c == 1)
        def _():
            run_pipe(CBASE[1] + s * CPT[1], CPT[1])

        plsc.subcore_barrier()

    with jax.named_scope("acc_out"):
        pltpu.sync_copy(acc_sh.at[pl.ds(s * NROWS_T, NROWS_T)],
                        out_hbm.at[c, pl.ds(s * NROWS_T, NROWS_T)])


def _prep_body(feat_ref, od_ref, wmu_ref, wrho_ref, weps_ref,
               bmu_ref, brho_ref, beps_ref, fs_ref, w_ref, b_ref):
    od = jnp.sum(od_ref[...], axis=0)
    od = jnp.maximum(od[:N], 1.0)
    fs_ref[pl.ds(0, N), :] = feat_ref[...] * lax.rsqrt(od)[:, None]
    fs_ref[pl.ds(N, NPAD - N), :] = jnp.zeros((NPAD - N, D), jnp.float32)
    w_ref[...] = wmu_ref[...] + weps_ref[...] * jnp.log1p(jnp.exp(wrho_ref[...]))
    b_ref[...] = bmu_ref[...] + beps_ref[...] * jnp.log1p(jnp.exp(brho_ref[...]))


_prep = pl.pallas_call(
    _prep_body,
    out_shape=(
        jax.ShapeDtypeStruct((NPAD, D), jnp.float32),
        jax.ShapeDtypeStruct((D, D), jnp.float32),
        jax.ShapeDtypeStruct((1, D), jnp.float32),
    ),
)


def _finish_body(p_ref, id_ref, w_ref, b_ref, out_ref):
    agg = p_ref[0] + p_ref[1]
    idg = jnp.maximum(jnp.sum(id_ref[...], axis=0)[:N], 1.0)
    rst = agg * lax.rsqrt(idg)[:, None]
    out_ref[...] = (
        jnp.dot(rst, w_ref[...], preferred_element_type=jnp.float32)
        + b_ref[...]
    )


_finish = pl.pallas_call(
    _finish_body,
    out_shape=jax.ShapeDtypeStruct((N, D), jnp.float32),
)


def kernel(feat, edge_index, W_mu, W_rho, bias_mu, bias_rho, W_eps, bias_eps):
    ei = edge_index.astype(jnp.int32)
    pad = N + jnp.arange(E_PAD - E, dtype=jnp.int32) % (NPAD - N)
    src_pad = jnp.concatenate([ei[0], pad])
    dst_pad = jnp.concatenate([ei[1], pad])

    degrees, aggregate = _sc_kernels()
    odeg_parts, ideg_parts = degrees(src_pad, dst_pad)
    feat_scaled, weight, bias = _prep(
        feat, odeg_parts, W_mu, W_rho, W_eps,
        bias_mu.reshape(1, D), bias_rho.reshape(1, D), bias_eps.reshape(1, D))
    zeros = jnp.zeros((NPAD, D), jnp.float32)
    partials = aggregate(feat_scaled,
                         src_pad.reshape(NW * NCH, CH),
                         dst_pad.reshape(NW * NCH, CH), zeros)
    return _finish(partials, ideg_parts, weight, bias)
